# baseline (speedup 1.0000x reference)
import jax
import jax.numpy as jnp
from jax.experimental import pallas as pl
from jax.experimental.pallas import tpu as pltpu


_TILE_I = 2048


def _split15(x, flip):
    hi = x.astype(jnp.bfloat16)
    lo = (x - hi.astype(jnp.float32)).astype(jnp.bfloat16)
    if flip:
        return jnp.concatenate([hi, lo, hi], axis=0)
    return jnp.concatenate([hi, hi, lo], axis=0)


def _hausdorff_kernel(a_ref, b_ref, out_ref, dist2_ref, m1_ref):
    b = pl.program_id(0)
    i = pl.program_id(1)
    ni = pl.num_programs(1)

    @pl.when(i == 0)
    def _init_batch():
        m1_ref[0, 0] = -jnp.inf
        dist2_ref[0:1, :] = jnp.full((1, dist2_ref.shape[1]), jnp.inf,
                                     dtype=jnp.float32)

    lhs15 = _split15(a_ref[0], flip=False)
    rhs15 = _split15(b_ref[0], flip=True)

    d = jax.lax.dot_general(
        lhs15, rhs15, (((0,), (0,)), ((), ())),
        preferred_element_type=jnp.float32)

    row_min = jnp.min(d, axis=1)
    m1_ref[0, 0] = jnp.maximum(m1_ref[0, 0], jnp.max(row_min))

    col_min = jnp.min(d, axis=0, keepdims=True)
    dist2_ref[0:1, :] = jnp.minimum(dist2_ref[0:1, :], col_min)

    @pl.when(i == ni - 1)
    def _finish_batch():
        m2 = jnp.max(dist2_ref[0:1, :])
        out_ref[0:1, 0:1, 0:1] = jnp.full((1, 1, 1),
                                          jnp.maximum(m1_ref[0, 0], m2))


def kernel(points1, points2):
    bsz, n1, _ = points1.shape
    _, n2, _ = points2.shape

    pts = jnp.concatenate([points1, points2], axis=0)
    nn = jnp.sum(pts * pts, axis=2, keepdims=True)
    ones = jnp.ones_like(nn)
    is_lhs = (jnp.arange(2 * bsz) < bsz).reshape(-1, 1, 1)
    aug = jnp.concatenate(
        [jnp.where(is_lhs, pts, -2.0 * pts),
         jnp.where(is_lhs, nn, ones),
         jnp.where(is_lhs, ones, nn)], axis=2)
    ab5 = jnp.transpose(aug, (0, 2, 1))

    ni = n1 // _TILE_I
    out = pl.pallas_call(
        _hausdorff_kernel,
        grid=(bsz, ni),
        in_specs=[
            pl.BlockSpec((1, 5, _TILE_I), lambda b, i: (b, 0, i)),
            pl.BlockSpec((1, 5, n2), lambda b, i, bsz=bsz: (b + bsz, 0, 0)),
        ],
        out_specs=pl.BlockSpec((1, 1, 1), lambda b, i: (b, 0, 0)),
        out_shape=jax.ShapeDtypeStruct((bsz, 1, 1), jnp.float32),
        scratch_shapes=[
            pltpu.VMEM((1, n2), jnp.float32),
            pltpu.SMEM((1, 1), jnp.float32),
        ],
        compiler_params=pltpu.CompilerParams(
            dimension_semantics=("parallel", "arbitrary")),
    )(ab5, ab5)
    return jnp.sum(out)


# speedup vs baseline: 1.0228x; 1.0228x over previous
import jax
import jax.numpy as jnp
from jax.experimental import pallas as pl
from jax.experimental.pallas import tpu as pltpu


_TILE_I = 2048


def _split15(x, flip):
    hi = x.astype(jnp.bfloat16)
    lo = (x - hi.astype(jnp.float32)).astype(jnp.bfloat16)
    if flip:
        return jnp.concatenate([hi, lo, hi], axis=0)
    return jnp.concatenate([hi, hi, lo], axis=0)


def _hausdorff_kernel(a_ref, b_ref, out_ref, rhs_ref, dist2_ref, m1_ref):
    b = pl.program_id(0)
    i = pl.program_id(1)
    ni = pl.num_programs(1)

    @pl.when(jnp.logical_and(b == 0, i == 0))
    def _init_out():
        out_ref[0:1, 0:1] = jnp.zeros((1, 1), jnp.float32)

    @pl.when(i == 0)
    def _init_batch():
        m1_ref[0, 0] = -jnp.inf
        dist2_ref[0:1, :] = jnp.full((1, dist2_ref.shape[1]), jnp.inf,
                                     dtype=jnp.float32)
        rhs_ref[:, :] = _split15(b_ref[0], flip=True)

    lhs15 = _split15(a_ref[0], flip=False)

    d = jax.lax.dot_general(
        lhs15, rhs_ref[:, :], (((0,), (0,)), ((), ())),
        preferred_element_type=jnp.float32)

    row_min = jnp.min(d, axis=1)
    m1_ref[0, 0] = jnp.maximum(m1_ref[0, 0], jnp.max(row_min))

    col_min = jnp.min(d, axis=0, keepdims=True)
    dist2_ref[0:1, :] = jnp.minimum(dist2_ref[0:1, :], col_min)

    @pl.when(i == ni - 1)
    def _finish_batch():
        m2 = jnp.max(dist2_ref[0:1, :])
        out_ref[0:1, 0:1] = out_ref[0:1, 0:1] + jnp.maximum(m1_ref[0, 0], m2)


def kernel(points1, points2):
    bsz, n1, _ = points1.shape
    _, n2, _ = points2.shape

    pts = jnp.concatenate([points1, points2], axis=0)
    nn = jnp.sum(pts * pts, axis=2, keepdims=True)
    ones = jnp.ones_like(nn)
    is_lhs = (jnp.arange(2 * bsz) < bsz).reshape(-1, 1, 1)
    aug = jnp.concatenate(
        [jnp.where(is_lhs, pts, -2.0 * pts),
         jnp.where(is_lhs, nn, ones),
         jnp.where(is_lhs, ones, nn)], axis=2)
    ab5 = jnp.transpose(aug, (0, 2, 1))

    ni = n1 // _TILE_I
    out = pl.pallas_call(
        _hausdorff_kernel,
        grid=(bsz, ni),
        in_specs=[
            pl.BlockSpec((1, 5, _TILE_I), lambda b, i: (b, 0, i)),
            pl.BlockSpec((1, 5, n2), lambda b, i, bsz=bsz: (b + bsz, 0, 0)),
        ],
        out_specs=pl.BlockSpec((1, 1), lambda b, i: (0, 0)),
        out_shape=jax.ShapeDtypeStruct((1, 1), jnp.float32),
        scratch_shapes=[
            pltpu.VMEM((15, n2), jnp.bfloat16),
            pltpu.VMEM((1, n2), jnp.float32),
            pltpu.SMEM((1, 1), jnp.float32),
        ],
    )(ab5, ab5)
    return out[0, 0]


# speedup vs baseline: 1.0788x; 1.0547x over previous
import jax
import jax.numpy as jnp
from jax.experimental import pallas as pl
from jax.experimental.pallas import tpu as pltpu


_CHUNK_I = 1024


def _split15(x, flip):
    hi = x.astype(jnp.bfloat16)
    lo = (x - hi.astype(jnp.float32)).astype(jnp.bfloat16)
    if flip:
        return jnp.concatenate([hi, lo, hi], axis=0)
    return jnp.concatenate([hi, hi, lo], axis=0)


def _hausdorff_kernel(a_ref, b_ref, out_ref):
    b = pl.program_id(0)

    @pl.when(b == 0)
    def _init_out():
        out_ref[0:1, 0:1] = jnp.zeros((1, 1), jnp.float32)

    lhs15 = _split15(a_ref[0], flip=False)
    rhs15 = _split15(b_ref[0], flip=True)

    n1 = lhs15.shape[1]
    m1 = None
    dist2 = None
    for k in range(n1 // _CHUNK_I):
        d = jax.lax.dot_general(
            lhs15[:, k * _CHUNK_I:(k + 1) * _CHUNK_I], rhs15,
            (((0,), (0,)), ((), ())),
            preferred_element_type=jnp.float32)
        m1_k = jnp.max(jnp.min(d, axis=1))
        m1 = m1_k if m1 is None else jnp.maximum(m1, m1_k)
        c_k = jnp.min(d, axis=0, keepdims=True)
        dist2 = c_k if dist2 is None else jnp.minimum(dist2, c_k)

    m = jnp.maximum(m1, jnp.max(dist2))
    out_ref[0:1, 0:1] = out_ref[0:1, 0:1] + m


def kernel(points1, points2):
    bsz, n1, _ = points1.shape
    _, n2, _ = points2.shape

    pts = jnp.concatenate([points1, points2], axis=0)
    nn = jnp.sum(pts * pts, axis=2, keepdims=True)
    ones = jnp.ones_like(nn)
    is_lhs = (jnp.arange(2 * bsz) < bsz).reshape(-1, 1, 1)
    aug = jnp.concatenate(
        [jnp.where(is_lhs, pts, -2.0 * pts),
         jnp.where(is_lhs, nn, ones),
         jnp.where(is_lhs, ones, nn)], axis=2)
    ab5 = jnp.transpose(aug, (0, 2, 1))

    out = pl.pallas_call(
        _hausdorff_kernel,
        grid=(bsz,),
        in_specs=[
            pl.BlockSpec((1, 5, n1), lambda b: (b, 0, 0)),
            pl.BlockSpec((1, 5, n2), lambda b, bsz=bsz: (b + bsz, 0, 0)),
        ],
        out_specs=pl.BlockSpec((1, 1), lambda b: (0, 0)),
        out_shape=jax.ShapeDtypeStruct((1, 1), jnp.float32),
    )(ab5, ab5)
    return out[0, 0]
